# baseline (speedup 1.0000x reference)
import functools

import jax
import jax.numpy as jnp
from jax import lax
from jax.experimental import pallas as pl
from jax.experimental.pallas import tpu as pltpu
from jax.experimental.pallas import tpu_sc as plsc

N_FIELDS = 26
VOCAB = 100000
DIM = 32
EPS = 1e-5
OUT_D = N_FIELDS * DIM

L = 16
NC = 2
NS = 16
NW = NC * NS

C = 64
CE = C * N_FIELDS
NIDX = CE // 128
NVEC = CE // L
OFF_P = 208


def _rsqrt_vec(x):
    i = plsc.bitcast(x, jnp.int32)
    i = jnp.int32(0x5F3759DF) - lax.shift_right_logical(i, 1)
    y = plsc.bitcast(i, jnp.float32)
    for _ in range(3):
        y = y * (1.5 - 0.5 * x * y * y)
    return y


def _make_sc_call(B):
    assert B % (NW * C) == 0
    chunks = B // (NW * C)
    mesh = plsc.VectorSubcoreMesh(core_axis_name="c", subcore_axis_name="s")

    @functools.partial(
        pl.kernel,
        mesh=mesh,
        compiler_params=pltpu.CompilerParams(
            needs_layout_passes=False, use_tc_tiling_on_sc=False),
        out_type=jax.ShapeDtypeStruct((B * N_FIELDS, DIM), jnp.float32),
        scratch_types=[
            pltpu.VMEM((CE,), jnp.int32),
            pltpu.VMEM((NIDX, 128), jnp.int32),
            pltpu.VMEM((CE, DIM), jnp.float32),
            pltpu.VMEM((OFF_P,), jnp.int32),
            pltpu.VMEM((OUT_D,), jnp.float32),
            pltpu.VMEM((OUT_D,), jnp.float32),
            pltpu.SemaphoreType.DMA,
        ],
    )
    def sc_call(tab, catf, off, gamma, beta, out,
                cat_v, idx_v, rows_v, off_v, g_v, b_v, sem):
        wid = lax.axis_index("s") * NC + lax.axis_index("c")
        pltpu.sync_copy(off, off_v)
        pltpu.sync_copy(gamma, g_v)
        pltpu.sync_copy(beta, b_v)

        def chunk_body(c, carry):
            row0 = (wid * chunks + c) * C
            e0 = row0 * N_FIELDS
            pltpu.sync_copy(catf.at[pl.ds(e0, CE)], cat_v)
            for i in range(NVEC):
                v = cat_v[pl.ds(i * L, L)]
                v = jnp.minimum(jnp.maximum(v, 0), VOCAB - 1)
                v = v + off_v[pl.ds((i % 13) * L, L)]
                idx_v[i // 8, pl.ds((i % 8) * L, L)] = v
            copies = [
                pltpu.async_copy(tab.at[idx_v.at[j]],
                                 rows_v.at[pl.ds(j * 128, 128)], sem)
                for j in range(NIDX)
            ]
            for cp in copies:
                cp.wait()

            def ln_body(r, carry2):
                base = r * N_FIELDS
                acc = jnp.zeros((L,), jnp.float32)
                acc2 = jnp.zeros((L,), jnp.float32)
                for k in range(N_FIELDS):
                    for h in range(2):
                        v = rows_v[base + k, pl.ds(h * L, L)]
                        acc = acc + v
                        acc2 = acc2 + v * v
                s = jnp.sum(acc)
                s2 = jnp.sum(acc2)
                mean = s * (1.0 / OUT_D)
                var = s2 * (1.0 / OUT_D) - mean * mean
                mean_v = jnp.full((L,), mean, jnp.float32)
                rstd_v = _rsqrt_vec(jnp.full((L,), var + EPS, jnp.float32))
                for k in range(N_FIELDS):
                    for h in range(2):
                        col = pl.ds(h * L, L)
                        v = rows_v[base + k, col]
                        gs = g_v[pl.ds(k * DIM + h * L, L)]
                        bs = b_v[pl.ds(k * DIM + h * L, L)]
                        rows_v[base + k, col] = (v - mean_v) * rstd_v * gs + bs
                return carry2

            lax.fori_loop(0, C, ln_body, 0)
            pltpu.sync_copy(rows_v, out.at[pl.ds(e0, CE)])
            return carry

        lax.fori_loop(0, chunks, chunk_body, 0)

    return sc_call


def kernel(cat, tables, gamma, beta):
    B = cat.shape[0]
    tab = tables.reshape(N_FIELDS * VOCAB, DIM)
    catf = cat.reshape(B * N_FIELDS)
    off = (jnp.arange(OFF_P, dtype=jnp.int32) % N_FIELDS) * VOCAB
    out = _make_sc_call(B)(tab, catf, off, gamma, beta)
    return out.reshape(B, OUT_D)
